# baseline (speedup 1.0000x reference)
import functools

import jax
import jax.numpy as jnp
from jax import lax
from jax.experimental import pallas as pl
from jax.experimental.pallas import tpu as pltpu
from jax.experimental.pallas import tpu_sc as plsc

B = 16384
D = 64
H = 128
OUT = 64

NC = 2
NS = 16
NW = NC * NS
B_PER_W = B // NW


def _sc_gather_both(user_table, product_table, user_ids, product_ids):
  mesh = plsc.VectorSubcoreMesh(core_axis_name="c", subcore_axis_name="s")

  @functools.partial(
      pl.kernel,
      mesh=mesh,
      compiler_params=pltpu.CompilerParams(use_tc_tiling_on_sc=False),
      out_type=(
          jax.ShapeDtypeStruct((B, D), jnp.float32),
          jax.ShapeDtypeStruct((B, D), jnp.float32),
      ),
      scratch_types=[
          pltpu.VMEM((B_PER_W,), jnp.int32),
          pltpu.VMEM((B_PER_W, D), jnp.float32),
          pltpu.VMEM((B_PER_W,), jnp.int32),
          pltpu.VMEM((B_PER_W, D), jnp.float32),
          pltpu.SemaphoreType.DMA,
          pltpu.SemaphoreType.DMA,
      ],
  )
  def k(utab_hbm, ptab_hbm, uid_hbm, pid_hbm, uout_hbm, pout_hbm,
        uidx_v, urows_v, pidx_v, prows_v, usem, psem):
    wid = lax.axis_index("s") * NC + lax.axis_index("c")
    base = wid * B_PER_W
    pltpu.sync_copy(uid_hbm.at[pl.ds(base, B_PER_W)], uidx_v)
    pltpu.sync_copy(pid_hbm.at[pl.ds(base, B_PER_W)], pidx_v)
    ug = pltpu.async_copy(utab_hbm.at[uidx_v], urows_v, usem)
    pg = pltpu.async_copy(ptab_hbm.at[pidx_v], prows_v, psem)
    ug.wait()
    pltpu.sync_copy(urows_v, uout_hbm.at[pl.ds(base, B_PER_W)])
    pg.wait()
    pltpu.sync_copy(prows_v, pout_hbm.at[pl.ds(base, B_PER_W)])

  return k(user_table, product_table, user_ids, product_ids)


BM = 2048


def _tc_mlp_body(u_ref, p_ref, wq1, bq1, wq2, bq2, wc1, bc1, wc2, bc2,
                 q_ref, c_ref):
  q = jnp.maximum(
      jnp.dot(u_ref[...], wq1[...], preferred_element_type=jnp.float32)
      + bq1[...], 0.0)
  q_ref[...] = jnp.maximum(
      jnp.dot(q, wq2[...], preferred_element_type=jnp.float32)
      + bq2[...], 0.0)
  c = jnp.maximum(
      jnp.dot(p_ref[...], wc1[...], preferred_element_type=jnp.float32)
      + bc1[...], 0.0)
  c_ref[...] = jnp.maximum(
      jnp.dot(c, wc2[...], preferred_element_type=jnp.float32)
      + bc2[...], 0.0)


def _tc_towers(pooled_u, pooled_p, Wq1, bq1, Wq2, bq2, Wc1, bc1, Wc2, bc2):
  full = lambda shape: pl.BlockSpec(shape, lambda i: (0, 0))
  return pl.pallas_call(
      _tc_mlp_body,
      grid=(B // BM,),
      in_specs=[
          pl.BlockSpec((BM, D), lambda i: (i, 0)),
          pl.BlockSpec((BM, D), lambda i: (i, 0)),
          full((D, H)), full((1, H)), full((H, OUT)), full((1, OUT)),
          full((D, H)), full((1, H)), full((H, OUT)), full((1, OUT)),
      ],
      out_specs=[
          pl.BlockSpec((BM, OUT), lambda i: (i, 0)),
          pl.BlockSpec((BM, OUT), lambda i: (i, 0)),
      ],
      out_shape=[
          jax.ShapeDtypeStruct((B, OUT), jnp.float32),
          jax.ShapeDtypeStruct((B, OUT), jnp.float32),
      ],
  )(pooled_u, pooled_p,
    Wq1, bq1.reshape(1, H), Wq2, bq2.reshape(1, OUT),
    Wc1, bc1.reshape(1, H), Wc2, bc2.reshape(1, OUT))


@jax.jit
def kernel(user_ids, product_ids, user_table, product_table,
           Wq1, bq1, Wq2, bq2, Wc1, bc1, Wc2, bc2):
  pooled_u, pooled_p = _sc_gather_both(
      user_table, product_table, user_ids, product_ids)
  q, c = _tc_towers(pooled_u, pooled_p,
                    Wq1, bq1, Wq2, bq2, Wc1, bc1, Wc2, bc2)
  return (q, c)
